# baseline (speedup 1.0000x reference)
import functools

import jax
import jax.numpy as jnp
from jax import lax
from jax.experimental import pallas as pl
from jax.experimental.pallas import tpu as pltpu
from jax.experimental.pallas import tpu_sc as plsc

NUM_SEG = 512
CHUNK = 128
NC = 2
NS = 16
NW = NC * NS

TC_BLK = 1024
SC_ROWS = 79872


def _sc_partials(readout, batch, zeros, n_sc):
    _, d = readout.shape
    nfull = n_sc // CHUNK
    iters = (nfull + NW - 1) // NW
    rows_per_tile = NUM_SEG // NS

    mesh = plsc.VectorSubcoreMesh(core_axis_name="c", subcore_axis_name="s")

    nb = 3

    @functools.partial(
        pl.kernel,
        out_type=jax.ShapeDtypeStruct((NC, NUM_SEG, d), jnp.float32),
        mesh=mesh,
        scratch_types=[
            pltpu.VMEM((nb, CHUNK, d), jnp.float32),
            pltpu.VMEM((nb, CHUNK), jnp.int32),
            pltpu.VMEM_SHARED((NUM_SEG, d), jnp.float32),
            pltpu.SemaphoreType.DMA,
            pltpu.SemaphoreType.DMA,
            pltpu.SemaphoreType.DMA,
        ],
    )
    def k(readout_hbm, batch_hbm, zeros_hbm, out_hbm,
          rows_v, idx_v, acc_s, sem0, sem1, sem2):
        cid = lax.axis_index("c")
        sid = lax.axis_index("s")
        wid = cid * NS + sid
        sems = (sem0, sem1, sem2)

        def start(i, b):
            base = (i * NW + wid) * CHUNK
            pltpu.async_copy(batch_hbm.at[pl.ds(base, CHUNK)],
                             idx_v.at[b], sems[b])
            pltpu.async_copy(readout_hbm.at[pl.ds(base, CHUNK)],
                             rows_v.at[b], sems[b])

        def drain(i, b):
            base = (i * NW + wid) * CHUNK
            pltpu.make_async_copy(batch_hbm.at[pl.ds(base, CHUNK)],
                                  idx_v.at[b], sems[b]).wait()
            pltpu.make_async_copy(readout_hbm.at[pl.ds(base, CHUNK)],
                                  rows_v.at[b], sems[b]).wait()

        for b in range(nb):
            @pl.when(b * NW + wid < nfull)
            def _(b=b):
                start(b, b)

        pltpu.sync_copy(
            zeros_hbm.at[pl.ds(sid * rows_per_tile, rows_per_tile)],
            acc_s.at[pl.ds(sid * rows_per_tile, rows_per_tile)])
        plsc.subcore_barrier()

        def outer(j, carry):
            for b in range(nb):
                i = j * nb + b
                c = i * NW + wid

                @pl.when(c < nfull)
                def _(i=i, b=b):
                    drain(i, b)
                    pltpu.sync_copy(rows_v.at[b], acc_s.at[idx_v.at[b]],
                                    add=True)

                    @pl.when((i + nb) * NW + wid < nfull)
                    def _():
                        start(i + nb, b)

            return carry

        lax.fori_loop(0, (iters + nb - 1) // nb, outer, 0)

        plsc.subcore_barrier()

        pltpu.sync_copy(
            acc_s.at[pl.ds(sid * rows_per_tile, rows_per_tile)],
            out_hbm.at[cid, pl.ds(sid * rows_per_tile, rows_per_tile)])

    return k(readout, batch, zeros)


def _tc_partial(readout, batch, row0):
    n, d = readout.shape
    nblk = (n - row0 + TC_BLK - 1) // TC_BLK
    blk0 = row0 // TC_BLK
    W = 128

    def body(ids_smem, rows_ref, ids_ref, out_ref):
        i = pl.program_id(0)

        @pl.when(i == 0)
        def _():
            out_ref[...] = jnp.zeros_like(out_ref)

        gbase = row0 + i * TC_BLK
        lastvalid = jnp.minimum(TC_BLK - 1, n - 1 - gbase)
        w0 = (ids_smem[0, 0] // W) * W
        wl = ids_smem[0, lastvalid]
        nw = (wl - w0) // W + 1

        ids = ids_ref[0][None, :]
        validc = gbase + lax.broadcasted_iota(jnp.int32, (TC_BLK, 1), 0) < n
        rows = jnp.where(validc, rows_ref[...], 0.0)

        def win(j, carry):
            s = w0 + j * W
            segs = s + lax.broadcasted_iota(jnp.int32, (W, TC_BLK), 0)
            onehot = jnp.where(segs == ids, 1.0, 0.0)
            psum = lax.dot(onehot, rows, preferred_element_type=jnp.float32)
            out_ref[pl.ds(s, W), :] += psum
            return carry

        lax.fori_loop(0, nw, win, 0)

    return pl.pallas_call(
        body,
        grid=(nblk,),
        in_specs=[
            pl.BlockSpec((1, TC_BLK), lambda i: (0, blk0 + i),
                         memory_space=pltpu.SMEM),
            pl.BlockSpec((TC_BLK, d), lambda i: (blk0 + i, 0)),
            pl.BlockSpec((1, TC_BLK), lambda i: (0, blk0 + i)),
        ],
        out_specs=pl.BlockSpec((NUM_SEG, d), lambda i: (0, 0)),
        out_shape=jax.ShapeDtypeStruct((NUM_SEG, d), jnp.float32),
    )(batch.reshape(1, -1), readout, batch.reshape(1, -1))


def _combine(sc, tc):
    def body(p_ref, t_ref, o_ref):
        o_ref[...] = p_ref[0] + p_ref[1] + t_ref[...]

    return pl.pallas_call(
        body,
        out_shape=jax.ShapeDtypeStruct(tc.shape, tc.dtype),
    )(sc, tc)


def kernel(readout, batch):
    n, d = readout.shape
    n_sc = min(SC_ROWS, (n // CHUNK) * CHUNK)
    zeros = jnp.zeros((NUM_SEG, d), jnp.float32)
    batch = batch.astype(jnp.int32)
    sc = _sc_partials(readout, batch, zeros, n_sc)
    tc = _tc_partial(readout, batch, n_sc)
    return _combine(sc, tc)


# speedup vs baseline: 1.0182x; 1.0182x over previous
import functools

import jax
import jax.numpy as jnp
from jax import lax
from jax.experimental import pallas as pl
from jax.experimental.pallas import tpu as pltpu
from jax.experimental.pallas import tpu_sc as plsc

NUM_SEG = 512
CHUNK = 128
NC = 2
NS = 16
NW = NC * NS

TC_BLK = 1024
SC_ROWS = 75776


def _sc_partials(readout, batch, zeros, n_sc):
    _, d = readout.shape
    nfull = n_sc // CHUNK
    iters = (nfull + NW - 1) // NW
    rows_per_tile = NUM_SEG // NS

    mesh = plsc.VectorSubcoreMesh(core_axis_name="c", subcore_axis_name="s")

    nb = 3

    @functools.partial(
        pl.kernel,
        out_type=jax.ShapeDtypeStruct((NC, NUM_SEG, d), jnp.float32),
        mesh=mesh,
        scratch_types=[
            pltpu.VMEM((nb, CHUNK, d), jnp.float32),
            pltpu.VMEM((nb, CHUNK), jnp.int32),
            pltpu.VMEM_SHARED((NUM_SEG, d), jnp.float32),
            pltpu.SemaphoreType.DMA,
            pltpu.SemaphoreType.DMA,
            pltpu.SemaphoreType.DMA,
        ],
    )
    def k(readout_hbm, batch_hbm, zeros_hbm, out_hbm,
          rows_v, idx_v, acc_s, sem0, sem1, sem2):
        cid = lax.axis_index("c")
        sid = lax.axis_index("s")
        wid = cid * NS + sid
        sems = (sem0, sem1, sem2)

        def start(i, b):
            base = (i * NW + wid) * CHUNK
            pltpu.async_copy(batch_hbm.at[pl.ds(base, CHUNK)],
                             idx_v.at[b], sems[b])
            pltpu.async_copy(readout_hbm.at[pl.ds(base, CHUNK)],
                             rows_v.at[b], sems[b])

        def drain(i, b):
            base = (i * NW + wid) * CHUNK
            pltpu.make_async_copy(batch_hbm.at[pl.ds(base, CHUNK)],
                                  idx_v.at[b], sems[b]).wait()
            pltpu.make_async_copy(readout_hbm.at[pl.ds(base, CHUNK)],
                                  rows_v.at[b], sems[b]).wait()

        for b in range(nb):
            @pl.when(b * NW + wid < nfull)
            def _(b=b):
                start(b, b)

        pltpu.sync_copy(
            zeros_hbm.at[pl.ds(sid * rows_per_tile, rows_per_tile)],
            acc_s.at[pl.ds(sid * rows_per_tile, rows_per_tile)])
        plsc.subcore_barrier()

        def outer(j, carry):
            for b in range(nb):
                i = j * nb + b
                c = i * NW + wid

                @pl.when(c < nfull)
                def _(i=i, b=b):
                    drain(i, b)
                    pltpu.sync_copy(rows_v.at[b], acc_s.at[idx_v.at[b]],
                                    add=True)

                    @pl.when((i + nb) * NW + wid < nfull)
                    def _():
                        start(i + nb, b)

            return carry

        lax.fori_loop(0, (iters + nb - 1) // nb, outer, 0)

        plsc.subcore_barrier()

        pltpu.sync_copy(
            acc_s.at[pl.ds(sid * rows_per_tile, rows_per_tile)],
            out_hbm.at[cid, pl.ds(sid * rows_per_tile, rows_per_tile)])

    return k(readout, batch, zeros)


def _tc_partial(readout, batch, row0):
    n, d = readout.shape
    nblk = (n - row0 + TC_BLK - 1) // TC_BLK
    blk0 = row0 // TC_BLK
    W = 128

    def body(ids_smem, rows_ref, ids_ref, out_ref):
        i = pl.program_id(0)

        @pl.when(i == 0)
        def _():
            out_ref[...] = jnp.zeros_like(out_ref)

        gbase = row0 + i * TC_BLK
        lastvalid = jnp.minimum(TC_BLK - 1, n - 1 - gbase)
        w0 = (ids_smem[0, 0] // W) * W
        wl = ids_smem[0, lastvalid]
        nw = (wl - w0) // W + 1

        ids = ids_ref[0][None, :]
        validc = gbase + lax.broadcasted_iota(jnp.int32, (TC_BLK, 1), 0) < n
        rows = jnp.where(validc, rows_ref[...], 0.0)

        def win(j, carry):
            s = w0 + j * W
            segs = s + lax.broadcasted_iota(jnp.int32, (W, TC_BLK), 0)
            onehot = jnp.where(segs == ids, 1.0, 0.0)
            psum = lax.dot(onehot, rows, preferred_element_type=jnp.float32)
            out_ref[pl.ds(s, W), :] += psum
            return carry

        lax.fori_loop(0, nw, win, 0)

    return pl.pallas_call(
        body,
        grid=(nblk,),
        in_specs=[
            pl.BlockSpec((1, TC_BLK), lambda i: (0, blk0 + i),
                         memory_space=pltpu.SMEM),
            pl.BlockSpec((TC_BLK, d), lambda i: (blk0 + i, 0)),
            pl.BlockSpec((1, TC_BLK), lambda i: (0, blk0 + i)),
        ],
        out_specs=pl.BlockSpec((NUM_SEG, d), lambda i: (0, 0)),
        out_shape=jax.ShapeDtypeStruct((NUM_SEG, d), jnp.float32),
    )(batch.reshape(1, -1), readout, batch.reshape(1, -1))


def _combine(sc, tc):
    def body(p_ref, t_ref, o_ref):
        o_ref[...] = p_ref[0] + p_ref[1] + t_ref[...]

    return pl.pallas_call(
        body,
        out_shape=jax.ShapeDtypeStruct(tc.shape, tc.dtype),
    )(sc, tc)


def kernel(readout, batch):
    n, d = readout.shape
    n_sc = min(SC_ROWS, (n // CHUNK) * CHUNK)
    zeros = jnp.zeros((NUM_SEG, d), jnp.float32)
    batch = batch.astype(jnp.int32)
    sc = _sc_partials(readout, batch, zeros, n_sc)
    tc = _tc_partial(readout, batch, n_sc)
    return _combine(sc, tc)


# speedup vs baseline: 1.0243x; 1.0060x over previous
import functools

import jax
import jax.numpy as jnp
from jax import lax
from jax.experimental import pallas as pl
from jax.experimental.pallas import tpu as pltpu
from jax.experimental.pallas import tpu_sc as plsc

NUM_SEG = 512
CHUNK = 128
NC = 2
NS = 16
NW = NC * NS

TC_BLK = 1024
SC_ROWS = 77824


def _sc_partials(readout, batch, zeros, n_sc):
    _, d = readout.shape
    nfull = n_sc // CHUNK
    iters = (nfull + NW - 1) // NW
    rows_per_tile = NUM_SEG // NS

    mesh = plsc.VectorSubcoreMesh(core_axis_name="c", subcore_axis_name="s")

    nb = 3

    @functools.partial(
        pl.kernel,
        out_type=jax.ShapeDtypeStruct((NC, NUM_SEG, d), jnp.float32),
        mesh=mesh,
        scratch_types=[
            pltpu.VMEM((nb, CHUNK, d), jnp.float32),
            pltpu.VMEM((nb, CHUNK), jnp.int32),
            pltpu.VMEM_SHARED((NUM_SEG, d), jnp.float32),
            pltpu.SemaphoreType.DMA,
            pltpu.SemaphoreType.DMA,
            pltpu.SemaphoreType.DMA,
        ],
    )
    def k(readout_hbm, batch_hbm, zeros_hbm, out_hbm,
          rows_v, idx_v, acc_s, sem0, sem1, sem2):
        cid = lax.axis_index("c")
        sid = lax.axis_index("s")
        wid = cid * NS + sid
        sems = (sem0, sem1, sem2)

        def start(i, b):
            base = (i * NW + wid) * CHUNK
            pltpu.async_copy(batch_hbm.at[pl.ds(base, CHUNK)],
                             idx_v.at[b], sems[b])
            pltpu.async_copy(readout_hbm.at[pl.ds(base, CHUNK)],
                             rows_v.at[b], sems[b])

        def drain(i, b):
            base = (i * NW + wid) * CHUNK
            pltpu.make_async_copy(batch_hbm.at[pl.ds(base, CHUNK)],
                                  idx_v.at[b], sems[b]).wait()
            pltpu.make_async_copy(readout_hbm.at[pl.ds(base, CHUNK)],
                                  rows_v.at[b], sems[b]).wait()

        for b in range(nb):
            @pl.when(b * NW + wid < nfull)
            def _(b=b):
                start(b, b)

        pltpu.sync_copy(
            zeros_hbm.at[pl.ds(sid * rows_per_tile, rows_per_tile)],
            acc_s.at[pl.ds(sid * rows_per_tile, rows_per_tile)])
        plsc.subcore_barrier()

        def outer(j, carry):
            for b in range(nb):
                i = j * nb + b
                c = i * NW + wid

                @pl.when(c < nfull)
                def _(i=i, b=b):
                    drain(i, b)
                    pltpu.sync_copy(rows_v.at[b], acc_s.at[idx_v.at[b]],
                                    add=True)

                    @pl.when((i + nb) * NW + wid < nfull)
                    def _():
                        start(i + nb, b)

            return carry

        lax.fori_loop(0, (iters + nb - 1) // nb, outer, 0)

        plsc.subcore_barrier()

        pltpu.sync_copy(
            acc_s.at[pl.ds(sid * rows_per_tile, rows_per_tile)],
            out_hbm.at[cid, pl.ds(sid * rows_per_tile, rows_per_tile)])

    return k(readout, batch, zeros)


def _tc_partial(readout, batch, row0):
    n, d = readout.shape
    nblk = (n - row0 + TC_BLK - 1) // TC_BLK
    blk0 = row0 // TC_BLK
    W = 128

    def body(ids_smem, rows_ref, ids_ref, out_ref):
        i = pl.program_id(0)

        @pl.when(i == 0)
        def _():
            out_ref[...] = jnp.zeros_like(out_ref)

        gbase = row0 + i * TC_BLK
        lastvalid = jnp.minimum(TC_BLK - 1, n - 1 - gbase)
        w0 = (ids_smem[0, 0] // W) * W
        wl = ids_smem[0, lastvalid]
        nw = (wl - w0) // W + 1

        ids = ids_ref[0][None, :]
        validc = gbase + lax.broadcasted_iota(jnp.int32, (TC_BLK, 1), 0) < n
        rows = jnp.where(validc, rows_ref[...], 0.0)

        def win(j, carry):
            s = w0 + j * W
            segs = s + lax.broadcasted_iota(jnp.int32, (W, TC_BLK), 0)
            onehot = jnp.where(segs == ids, 1.0, 0.0)
            psum = lax.dot(onehot, rows, preferred_element_type=jnp.float32)
            out_ref[pl.ds(s, W), :] += psum
            return carry

        lax.fori_loop(0, nw, win, 0)

    return pl.pallas_call(
        body,
        grid=(nblk,),
        in_specs=[
            pl.BlockSpec((1, TC_BLK), lambda i: (0, blk0 + i),
                         memory_space=pltpu.SMEM),
            pl.BlockSpec((TC_BLK, d), lambda i: (blk0 + i, 0)),
            pl.BlockSpec((1, TC_BLK), lambda i: (0, blk0 + i)),
        ],
        out_specs=pl.BlockSpec((NUM_SEG, d), lambda i: (0, 0)),
        out_shape=jax.ShapeDtypeStruct((NUM_SEG, d), jnp.float32),
    )(batch.reshape(1, -1), readout, batch.reshape(1, -1))


def _combine(sc, tc):
    def body(p_ref, t_ref, o_ref):
        o_ref[...] = p_ref[0] + p_ref[1] + t_ref[...]

    return pl.pallas_call(
        body,
        out_shape=jax.ShapeDtypeStruct(tc.shape, tc.dtype),
    )(sc, tc)


def kernel(readout, batch):
    n, d = readout.shape
    n_sc = min(SC_ROWS, (n // CHUNK) * CHUNK)
    zeros = jnp.zeros((NUM_SEG, d), jnp.float32)
    batch = batch.astype(jnp.int32)
    sc = _sc_partials(readout, batch, zeros, n_sc)
    tc = _tc_partial(readout, batch, n_sc)
    return _combine(sc, tc)
